# Initial kernel scaffold; baseline (speedup 1.0000x reference)
import jax
import jax.numpy as jnp
from jax.experimental import pallas as pl


def kernel(trigger_edge_index, trigger_edge_weights, x, thrd):
    raise NotImplementedError("write your pallas kernel here")


# speedup vs baseline: 2.8936x; 2.8936x over previous
import functools

import jax
import jax.numpy as jnp
from jax import lax
from jax.experimental import pallas as pl
from jax.experimental.pallas import tpu as pltpu
from jax.experimental.pallas import tpu_sc as plsc

N_NODES = 10000
N_EDGES = 320000
D = 128
L = 16
NW = 32
EPW = N_EDGES // NW
CH = 80
NCH = EPW // CH
GPC = CH // L


def _normalize_rows(x):
    def body(x_ref, o_ref):
        xb = x_ref[...]
        n2 = jnp.sum(xb * xb, axis=1, keepdims=True)
        o_ref[...] = xb * lax.rsqrt(jnp.maximum(n2, 1e-16))

    return pl.pallas_call(
        body, out_shape=jax.ShapeDtypeStruct(x.shape, x.dtype)
    )(x)


@functools.partial(
    pl.kernel,
    out_type=jax.ShapeDtypeStruct((NW, 2, L), jnp.float32),
    mesh=plsc.VectorSubcoreMesh(core_axis_name="c", subcore_axis_name="s"),
    compiler_params=pltpu.CompilerParams(needs_layout_passes=False),
    scratch_types=[
        pltpu.VMEM((CH,), jnp.int32),
        pltpu.VMEM((CH,), jnp.int32),
        pltpu.VMEM((CH,), jnp.float32),
        pltpu.VMEM((CH, D), jnp.float32),
        pltpu.VMEM((CH, D), jnp.float32),
        pltpu.VMEM((L,), jnp.float32),
        pltpu.VMEM((L, L), jnp.float32),
        pltpu.VMEM((2, L), jnp.float32),
        pltpu.SemaphoreType.DMA,
        pltpu.SemaphoreType.DMA,
    ],
)
def _edge_loss_sc(src_hbm, dst_hbm, w_hbm, xn_hbm, thrd_hbm, out_hbm,
                  idx_s, idx_d, w_v, rows_s, rows_d, thrd_v, acc_scr, out_v,
                  sem0, sem1):
    wid = lax.axis_index("s") * 2 + lax.axis_index("c")
    pltpu.sync_copy(thrd_hbm, thrd_v)
    tv = thrd_v[...]
    lanes = lax.iota(jnp.int32, L)
    zero = jnp.zeros((L,), jnp.float32)

    def chunk_body(c, carry):
        loss_vec, cnt_vec = carry
        base = wid * EPW + c * CH
        pltpu.sync_copy(src_hbm.at[pl.ds(base, CH)], idx_s)
        pltpu.sync_copy(dst_hbm.at[pl.ds(base, CH)], idx_d)
        pltpu.sync_copy(w_hbm.at[pl.ds(base, CH)], w_v)
        cp0 = pltpu.async_copy(xn_hbm.at[idx_s], rows_s, sem0)
        cp1 = pltpu.async_copy(xn_hbm.at[idx_d], rows_d, sem1)
        cp0.wait()
        cp1.wait()

        def group_body(g, gcarry):
            loss_vec, cnt_vec = gcarry
            w_vec = w_v[pl.ds(g * L, L)]
            m_vec = jnp.where(w_vec > 0.0, 1.0, 0.0).astype(jnp.float32)
            for e in range(L):
                r = g * L + e
                acc = rows_s[r, pl.ds(0, L)] * rows_d[r, pl.ds(0, L)]
                for k in range(1, D // L):
                    acc = acc + rows_s[r, pl.ds(k * L, L)] * rows_d[r, pl.ds(k * L, L)]
                acc_scr[e, :] = acc
            sims = plsc.load_gather(acc_scr, [lanes, jnp.zeros((L,), jnp.int32)])
            for j in range(1, L):
                sims = sims + plsc.load_gather(
                    acc_scr, [lanes, jnp.full((L,), j, jnp.int32)])
            vals = jnp.maximum(tv - sims, 0.0) * m_vec
            return loss_vec + vals, cnt_vec + m_vec

        return lax.fori_loop(0, GPC, group_body, (loss_vec, cnt_vec))

    loss_vec, cnt_vec = lax.fori_loop(0, NCH, chunk_body, (zero, zero))
    out_v[0, :] = loss_vec
    out_v[1, :] = cnt_vec
    pltpu.sync_copy(out_v, out_hbm.at[wid])


def kernel(trigger_edge_index, trigger_edge_weights, x, thrd):
    xn = _normalize_rows(x)
    src = trigger_edge_index[0]
    dst = trigger_edge_index[1]
    thrd_vec = jnp.full((L,), thrd, jnp.float32)
    parts = _edge_loss_sc(src, dst, trigger_edge_weights, xn, thrd_vec)
    return jnp.sum(parts[:, 0, :]) / jnp.sum(parts[:, 1, :])


# speedup vs baseline: 6.4586x; 2.2320x over previous
import functools

import jax
import jax.numpy as jnp
from jax import lax
from jax.experimental import pallas as pl
from jax.experimental.pallas import tpu as pltpu
from jax.experimental.pallas import tpu_sc as plsc

N_NODES = 10000
N_EDGES = 320000
D = 128
L = 16
NW = 32
EPW = N_EDGES // NW
CH = 80
NCH = EPW // CH
GPC = CH // L


def _normalize_rows(x):
    def body(x_ref, o_ref):
        xb = x_ref[...]
        n2 = jnp.sum(xb * xb, axis=1, keepdims=True)
        o_ref[...] = xb * lax.rsqrt(jnp.maximum(n2, 1e-16))

    return pl.pallas_call(
        body, out_shape=jax.ShapeDtypeStruct(x.shape, x.dtype)
    )(x)


@functools.partial(
    pl.kernel,
    out_type=jax.ShapeDtypeStruct((NW, 2, L), jnp.float32),
    mesh=plsc.VectorSubcoreMesh(core_axis_name="c", subcore_axis_name="s"),
    compiler_params=pltpu.CompilerParams(needs_layout_passes=False),
    scratch_types=[
        pltpu.VMEM((EPW,), jnp.int32),
        pltpu.VMEM((EPW,), jnp.int32),
        pltpu.VMEM((EPW,), jnp.float32),
        pltpu.VMEM((CH, D), jnp.float32),
        pltpu.VMEM((CH, D), jnp.float32),
        pltpu.VMEM((CH, D), jnp.float32),
        pltpu.VMEM((CH, D), jnp.float32),
        pltpu.VMEM((L,), jnp.float32),
        pltpu.VMEM((L, L), jnp.float32),
        pltpu.VMEM((2, L), jnp.float32),
        pltpu.SemaphoreType.DMA,
        pltpu.SemaphoreType.DMA,
    ],
)
def _edge_loss_sc(src_hbm, dst_hbm, w_hbm, xn_hbm, thrd_hbm, out_hbm,
                  idx_s, idx_d, w_v, rs_a, rd_a, rs_b, rd_b,
                  thrd_v, acc_scr, out_v, sem_a, sem_b):
    wid = lax.axis_index("s") * 2 + lax.axis_index("c")
    base = wid * EPW
    pltpu.sync_copy(thrd_hbm, thrd_v)
    pltpu.sync_copy(src_hbm.at[pl.ds(base, EPW)], idx_s)
    pltpu.sync_copy(dst_hbm.at[pl.ds(base, EPW)], idx_d)
    pltpu.sync_copy(w_hbm.at[pl.ds(base, EPW)], w_v)
    tv = thrd_v[...]
    zero = jnp.zeros((L,), jnp.float32)
    lanes = lax.iota(jnp.int32, L)

    def issue(c, rs, rd, sem):
        pltpu.async_copy(xn_hbm.at[idx_s.at[pl.ds(c * CH, CH)]], rs, sem)
        pltpu.async_copy(xn_hbm.at[idx_d.at[pl.ds(c * CH, CH)]], rd, sem)

    def drain(rs, rd, sem):
        pltpu.make_async_copy(xn_hbm.at[pl.ds(0, CH)], rs, sem).wait()
        pltpu.make_async_copy(xn_hbm.at[pl.ds(0, CH)], rd, sem).wait()

    def compute(c, rows_s, rows_d, carry):
        loss_vec, cnt_vec = carry

        def group_body(g, gcarry):
            loss_vec, cnt_vec = gcarry
            w_vec = w_v[pl.ds(c * CH + g * L, L)]
            m_vec = jnp.where(w_vec > 0.0, 1.0, 0.0).astype(jnp.float32)
            for e in range(L):
                r = g * L + e
                acc = rows_s[r, pl.ds(0, L)] * rows_d[r, pl.ds(0, L)]
                for k in range(1, D // L):
                    acc = acc + rows_s[r, pl.ds(k * L, L)] * rows_d[r, pl.ds(k * L, L)]
                acc_scr[e, :] = acc
            sims = plsc.load_gather(acc_scr, [lanes, jnp.zeros((L,), jnp.int32)])
            for j in range(1, L):
                sims = sims + plsc.load_gather(
                    acc_scr, [lanes, jnp.full((L,), j, jnp.int32)])
            vals = jnp.maximum(tv - sims, 0.0) * m_vec
            return loss_vec + vals, cnt_vec + m_vec

        return lax.fori_loop(0, GPC, group_body, (loss_vec, cnt_vec))

    issue(0, rs_a, rd_a, sem_a)
    issue(1, rs_b, rd_b, sem_b)

    def pair_body(i, carry):
        c0 = 2 * i
        drain(rs_a, rd_a, sem_a)
        carry = compute(c0, rs_a, rd_a, carry)

        @pl.when(c0 + 2 < NCH)
        def _():
            issue(c0 + 2, rs_a, rd_a, sem_a)

        c1 = 2 * i + 1
        drain(rs_b, rd_b, sem_b)
        carry = compute(c1, rs_b, rd_b, carry)

        @pl.when(c1 + 2 < NCH)
        def _():
            issue(c1 + 2, rs_b, rd_b, sem_b)

        return carry

    carry = lax.fori_loop(0, NCH // 2, pair_body, (zero, zero))
    drain(rs_a, rd_a, sem_a)
    loss_vec, cnt_vec = compute(NCH - 1, rs_a, rd_a, carry)

    out_v[0, :] = loss_vec
    out_v[1, :] = cnt_vec
    pltpu.sync_copy(out_v, out_hbm.at[wid])


def kernel(trigger_edge_index, trigger_edge_weights, x, thrd):
    xn = _normalize_rows(x)
    src = trigger_edge_index[0]
    dst = trigger_edge_index[1]
    thrd_vec = jnp.full((L,), thrd, jnp.float32)
    parts = _edge_loss_sc(src, dst, trigger_edge_weights, xn, thrd_vec)
    return jnp.sum(parts[:, 0, :]) / jnp.sum(parts[:, 1, :])


# speedup vs baseline: 6.5668x; 1.0168x over previous
import functools

import jax
import jax.numpy as jnp
from jax import lax
from jax.experimental import pallas as pl
from jax.experimental.pallas import tpu as pltpu
from jax.experimental.pallas import tpu_sc as plsc

N_NODES = 10000
N_EDGES = 320000
D = 128
DW = D // 2
L = 16
NW = 32
EPW = N_EDGES // NW
CH = 80
NCH = EPW // CH
GPC = CH // L


def _normalize_rows(x):
    def body(x_ref, o_ref):
        xb = x_ref[...]
        n2 = jnp.sum(xb * xb, axis=1, keepdims=True)
        o_ref[...] = (xb * lax.rsqrt(jnp.maximum(n2, 1e-16))).astype(
            jnp.bfloat16)

    return pl.pallas_call(
        body, out_shape=jax.ShapeDtypeStruct(x.shape, jnp.bfloat16)
    )(x)


@functools.partial(
    pl.kernel,
    out_type=jax.ShapeDtypeStruct((NW, 2, L), jnp.float32),
    mesh=plsc.VectorSubcoreMesh(core_axis_name="c", subcore_axis_name="s"),
    compiler_params=pltpu.CompilerParams(
        needs_layout_passes=False, use_tc_tiling_on_sc=False),
    scratch_types=[
        pltpu.VMEM((EPW,), jnp.int32),
        pltpu.VMEM((EPW,), jnp.int32),
        pltpu.VMEM((EPW,), jnp.float32),
        pltpu.VMEM((CH, DW), jnp.int32),
        pltpu.VMEM((CH, DW), jnp.int32),
        pltpu.VMEM((CH, DW), jnp.int32),
        pltpu.VMEM((CH, DW), jnp.int32),
        pltpu.VMEM((L,), jnp.float32),
        pltpu.VMEM((L, L), jnp.float32),
        pltpu.VMEM((2, L), jnp.float32),
        pltpu.SemaphoreType.DMA,
        pltpu.SemaphoreType.DMA,
    ],
)
def _edge_loss_sc(src_hbm, dst_hbm, w_hbm, xn_hbm, thrd_hbm, out_hbm,
                  idx_s, idx_d, w_v, rs_a, rd_a, rs_b, rd_b,
                  thrd_v, acc_scr, out_v, sem_a, sem_b):
    wid = lax.axis_index("s") * 2 + lax.axis_index("c")
    base = wid * EPW
    pltpu.sync_copy(thrd_hbm, thrd_v)
    pltpu.sync_copy(src_hbm.at[pl.ds(base, EPW)], idx_s)
    pltpu.sync_copy(dst_hbm.at[pl.ds(base, EPW)], idx_d)
    pltpu.sync_copy(w_hbm.at[pl.ds(base, EPW)], w_v)
    tv = thrd_v[...]
    zero = jnp.zeros((L,), jnp.float32)
    lanes = lax.iota(jnp.int32, L)

    def issue(c, rs, rd, sem):
        pltpu.async_copy(xn_hbm.at[idx_s.at[pl.ds(c * CH, CH)]], rs, sem)
        pltpu.async_copy(xn_hbm.at[idx_d.at[pl.ds(c * CH, CH)]], rd, sem)

    def drain(rs, rd, sem):
        pltpu.make_async_copy(xn_hbm.at[pl.ds(0, CH)], rs, sem).wait()
        pltpu.make_async_copy(xn_hbm.at[pl.ds(0, CH)], rd, sem).wait()


    def compute(c, rows_s, rows_d, carry):
        loss_vec, cnt_vec = carry

        def group_body(g, gcarry):
            loss_vec, cnt_vec = gcarry
            w_vec = w_v[pl.ds(c * CH + g * L, L)]
            m_vec = jnp.where(w_vec > 0.0, 1.0, 0.0).astype(jnp.float32)
            for e in range(L):
                r = g * L + e
                acc = jnp.zeros((L,), jnp.float32)
                for k in range(DW // L):
                    a = plsc.bitcast(rows_s[r, pl.ds(k * L, L)], jnp.bfloat16)
                    b = plsc.bitcast(rows_d[r, pl.ds(k * L, L)], jnp.bfloat16)
                    plo, phi = plsc.unpack(
                        a * b, format=plsc.PackFormat.INTERLEAVED)
                    acc = acc + plo + phi
                acc_scr[e, :] = acc
            sims = plsc.load_gather(acc_scr, [lanes, jnp.zeros((L,), jnp.int32)])
            for j in range(1, L):
                sims = sims + plsc.load_gather(
                    acc_scr, [lanes, jnp.full((L,), j, jnp.int32)])
            vals = jnp.maximum(tv - sims, 0.0) * m_vec
            return loss_vec + vals, cnt_vec + m_vec

        return lax.fori_loop(0, GPC, group_body, (loss_vec, cnt_vec))

    issue(0, rs_a, rd_a, sem_a)
    issue(1, rs_b, rd_b, sem_b)

    def pair_body(i, carry):
        c0 = 2 * i
        drain(rs_a, rd_a, sem_a)
        carry = compute(c0, rs_a, rd_a, carry)

        @pl.when(c0 + 2 < NCH)
        def _():
            issue(c0 + 2, rs_a, rd_a, sem_a)

        c1 = 2 * i + 1
        drain(rs_b, rd_b, sem_b)
        carry = compute(c1, rs_b, rd_b, carry)

        @pl.when(c1 + 2 < NCH)
        def _():
            issue(c1 + 2, rs_b, rd_b, sem_b)

        return carry

    carry = lax.fori_loop(0, NCH // 2, pair_body, (zero, zero))
    drain(rs_a, rd_a, sem_a)
    loss_vec, cnt_vec = compute(NCH - 1, rs_a, rd_a, carry)

    out_v[0, :] = loss_vec
    out_v[1, :] = cnt_vec
    pltpu.sync_copy(out_v, out_hbm.at[wid])


def kernel(trigger_edge_index, trigger_edge_weights, x, thrd):
    xn_bf16 = _normalize_rows(x)
    xn = lax.bitcast_convert_type(
        xn_bf16.reshape(N_NODES, DW, 2), jnp.int32)
    src = trigger_edge_index[0]
    dst = trigger_edge_index[1]
    thrd_vec = jnp.full((L,), thrd, jnp.float32)
    parts = _edge_loss_sc(src, dst, trigger_edge_weights, xn, thrd_vec)
    return jnp.sum(parts[:, 0, :]) / jnp.sum(parts[:, 1, :])


# speedup vs baseline: 7.8613x; 1.1971x over previous
import functools

import jax
import jax.numpy as jnp
from jax import lax
from jax.experimental import pallas as pl
from jax.experimental.pallas import tpu as pltpu
from jax.experimental.pallas import tpu_sc as plsc

N_NODES = 10000
N_EDGES = 320000
D = 128
DW = D // 2
L = 16
NW = 32
EPW = N_EDGES // NW
CH = 80
NCH = EPW // CH
GPC = CH // L


def _normalize_rows(x):
    def body(x_ref, o_ref):
        xb = x_ref[...]
        n2 = jnp.sum(xb * xb, axis=1, keepdims=True)
        xn = xb * lax.rsqrt(jnp.maximum(n2, 1e-16))
        lo = lax.bitcast_convert_type(
            xn[:, :DW].astype(jnp.bfloat16), jnp.uint16).astype(jnp.uint32)
        hi = lax.bitcast_convert_type(
            xn[:, DW:].astype(jnp.bfloat16), jnp.uint16).astype(jnp.uint32)
        o_ref[...] = ((hi << 16) | lo).astype(jnp.int32)

    return pl.pallas_call(
        body, out_shape=jax.ShapeDtypeStruct((N_NODES, DW), jnp.int32)
    )(x)


@functools.partial(
    pl.kernel,
    out_type=jax.ShapeDtypeStruct((NW, 2, L), jnp.float32),
    mesh=plsc.VectorSubcoreMesh(core_axis_name="c", subcore_axis_name="s"),
    compiler_params=pltpu.CompilerParams(
        needs_layout_passes=False, use_tc_tiling_on_sc=False),
    scratch_types=[
        pltpu.VMEM((EPW,), jnp.int32),
        pltpu.VMEM((EPW,), jnp.int32),
        pltpu.VMEM((EPW,), jnp.float32),
        pltpu.VMEM((CH, DW), jnp.int32),
        pltpu.VMEM((CH, DW), jnp.int32),
        pltpu.VMEM((CH, DW), jnp.int32),
        pltpu.VMEM((CH, DW), jnp.int32),
        pltpu.VMEM((L,), jnp.float32),
        pltpu.VMEM((L, L), jnp.float32),
        pltpu.VMEM((2, L), jnp.float32),
        pltpu.SemaphoreType.DMA,
        pltpu.SemaphoreType.DMA,
    ],
)
def _edge_loss_sc(src_hbm, dst_hbm, w_hbm, xn_hbm, thrd_hbm, out_hbm,
                  idx_s, idx_d, w_v, rs_a, rd_a, rs_b, rd_b,
                  thrd_v, acc_scr, out_v, sem_a, sem_b):
    wid = lax.axis_index("s") * 2 + lax.axis_index("c")
    base = wid * EPW
    pltpu.sync_copy(thrd_hbm, thrd_v)
    pltpu.sync_copy(src_hbm.at[pl.ds(base, EPW)], idx_s)
    pltpu.sync_copy(dst_hbm.at[pl.ds(base, EPW)], idx_d)
    pltpu.sync_copy(w_hbm.at[pl.ds(base, EPW)], w_v)
    tv = thrd_v[...]
    zero = jnp.zeros((L,), jnp.float32)
    lanes = lax.iota(jnp.int32, L)

    def issue(c, rs, rd, sem):
        pltpu.async_copy(xn_hbm.at[idx_s.at[pl.ds(c * CH, CH)]], rs, sem)
        pltpu.async_copy(xn_hbm.at[idx_d.at[pl.ds(c * CH, CH)]], rd, sem)

    def drain(rs, rd, sem):
        pltpu.make_async_copy(xn_hbm.at[pl.ds(0, CH)], rs, sem).wait()
        pltpu.make_async_copy(xn_hbm.at[pl.ds(0, CH)], rd, sem).wait()


    def compute(c, rows_s, rows_d, carry):
        loss_vec, cnt_vec = carry
        for g in range(GPC):
            w_vec = w_v[pl.ds(c * CH + g * L, L)]
            m_vec = jnp.where(w_vec > 0.0, 1.0, 0.0).astype(jnp.float32)
            for e in range(L):
                r = g * L + e
                parts = []
                for k in range(DW // L):
                    a = plsc.bitcast(rows_s[r, pl.ds(k * L, L)], jnp.bfloat16)
                    b = plsc.bitcast(rows_d[r, pl.ds(k * L, L)], jnp.bfloat16)
                    plo, phi = plsc.unpack(
                        a * b, format=plsc.PackFormat.INTERLEAVED)
                    parts.append(plo)
                    parts.append(phi)
                while len(parts) > 1:
                    parts = [parts[i] + parts[i + 1]
                             for i in range(0, len(parts) - 1, 2)] + (
                                 [parts[-1]] if len(parts) % 2 else [])
                acc_scr[e, :] = parts[0]
            cols = [
                plsc.load_gather(acc_scr, [lanes, jnp.full((L,), j, jnp.int32)])
                for j in range(L)
            ]
            while len(cols) > 1:
                cols = [cols[i] + cols[i + 1] for i in range(0, len(cols), 2)]
            sims = cols[0]
            vals = jnp.maximum(tv - sims, 0.0) * m_vec
            loss_vec = loss_vec + vals
            cnt_vec = cnt_vec + m_vec
        return loss_vec, cnt_vec

    issue(0, rs_a, rd_a, sem_a)
    issue(1, rs_b, rd_b, sem_b)

    def pair_body(i, carry):
        c0 = 2 * i
        drain(rs_a, rd_a, sem_a)
        carry = compute(c0, rs_a, rd_a, carry)

        @pl.when(c0 + 2 < NCH)
        def _():
            issue(c0 + 2, rs_a, rd_a, sem_a)

        c1 = 2 * i + 1
        drain(rs_b, rd_b, sem_b)
        carry = compute(c1, rs_b, rd_b, carry)

        @pl.when(c1 + 2 < NCH)
        def _():
            issue(c1 + 2, rs_b, rd_b, sem_b)

        return carry

    carry = lax.fori_loop(0, NCH // 2, pair_body, (zero, zero))
    drain(rs_a, rd_a, sem_a)
    loss_vec, cnt_vec = compute(NCH - 1, rs_a, rd_a, carry)

    out_v[0, :] = loss_vec
    out_v[1, :] = cnt_vec
    pltpu.sync_copy(out_v, out_hbm.at[wid])


def kernel(trigger_edge_index, trigger_edge_weights, x, thrd):
    xn = _normalize_rows(x)
    src = trigger_edge_index[0]
    dst = trigger_edge_index[1]
    thrd_vec = jnp.full((L,), thrd, jnp.float32)
    parts = _edge_loss_sc(src, dst, trigger_edge_weights, xn, thrd_vec)
    return jnp.sum(parts[:, 0, :]) / jnp.sum(parts[:, 1, :])
